# baseline (speedup 1.0000x reference)
import jax
import jax.numpy as jnp
from jax.experimental import pallas as pl

D = 2048
SK = 8
NSCORE = SK * SK
TOP_K = 8
BLK = 1024


def _router_kernel(x_ref, w_ref, idx_ref, probs_ref, scores_ref):
    x = x_ref[...]
    w = w_ref[...]
    s = jax.lax.dot_general(
        x, w, (((1,), (1,)), ((), ())),
        preferred_element_type=jnp.float32)
    scores = s[:, :NSCORE] + s[:, NSCORE:]
    scores_ref[...] = scores

    lane = jax.lax.broadcasted_iota(jnp.int32, (BLK, NSCORE), 1)
    rev = (NSCORE - 1 - lane).astype(jnp.float32)
    lane8 = jax.lax.broadcasted_iota(jnp.int32, (BLK, TOP_K), 1)
    vals = jnp.zeros((BLK, TOP_K), jnp.float32)
    revs = jnp.zeros((BLK, TOP_K), jnp.float32)
    cur = scores
    neg_one = jnp.float32(-1.0)
    neg_inf = jnp.float32(-jnp.inf)
    for k in range(TOP_K):
        m = jnp.max(cur, axis=-1, keepdims=True)
        sel = cur == m
        mi = jnp.max(jnp.where(sel, rev, neg_one), axis=-1,
                     keepdims=True)
        vals = jnp.where(lane8 == k, m, vals)
        revs = jnp.where(lane8 == k, mi, revs)
        cur = jnp.where(sel & (rev == mi), neg_inf, cur)

    idx_ref[...] = (NSCORE - 1) - revs.astype(jnp.int32)
    e = jnp.exp(vals - vals[:, :1])
    probs_ref[...] = e / jnp.sum(e, axis=-1, keepdims=True)


@jax.jit
def kernel(x, W1, W2):
    n_tok = x.shape[0]
    wr = jnp.concatenate(
        [jnp.repeat(W1, SK, axis=0), jnp.tile(W2, (SK, 1))], axis=0)
    grid = (n_tok // BLK,)
    out = pl.pallas_call(
        _router_kernel,
        grid=grid,
        in_specs=[
            pl.BlockSpec((BLK, D), lambda i: (i, 0)),
            pl.BlockSpec((2 * NSCORE, D), lambda i: (0, 0)),
        ],
        out_specs=[
            pl.BlockSpec((BLK, TOP_K), lambda i: (i, 0)),
            pl.BlockSpec((BLK, TOP_K), lambda i: (i, 0)),
            pl.BlockSpec((BLK, NSCORE), lambda i: (i, 0)),
        ],
        out_shape=[
            jax.ShapeDtypeStruct((n_tok, TOP_K), jnp.int32),
            jax.ShapeDtypeStruct((n_tok, TOP_K), jnp.float32),
            jax.ShapeDtypeStruct((n_tok, NSCORE), jnp.float32),
        ],
    )(x, wr)
    return (out[0], out[1], out[2])


# speedup vs baseline: 1.6798x; 1.6798x over previous
import jax
import jax.numpy as jnp
from jax.experimental import pallas as pl

D = 2048
SK = 8
NSCORE = SK * SK
TOP_K = 8
BLK = 1024


def _router_kernel(x_ref, w_ref, idx_ref, probs_ref, scores_ref):
    x = x_ref[...]
    w = w_ref[...]
    s = jax.lax.dot_general(
        x, w, (((1,), (1,)), ((), ())),
        preferred_element_type=jnp.float32)
    st = s.T
    s1t = st[:SK]
    s2t = st[SK:]
    cur = (s1t[:, None, :] + s2t[None, :, :]).reshape(NSCORE, BLK)
    scores_ref[...] = cur.T

    row = jax.lax.broadcasted_iota(jnp.int32, (NSCORE, BLK), 0)
    rev = (NSCORE - 1 - row).astype(jnp.float32)
    row8 = jax.lax.broadcasted_iota(jnp.int32, (TOP_K, BLK), 0)
    vals = jnp.zeros((TOP_K, BLK), jnp.float32)
    revs = jnp.zeros((TOP_K, BLK), jnp.float32)
    neg_one = jnp.float32(-1.0)
    neg_inf = jnp.float32(-jnp.inf)
    for k in range(TOP_K):
        m = jnp.max(cur, axis=0, keepdims=True)
        sel = cur == m
        mi = jnp.max(jnp.where(sel, rev, neg_one), axis=0,
                     keepdims=True)
        vals = jnp.where(row8 == k, m, vals)
        revs = jnp.where(row8 == k, mi, revs)
        cur = jnp.where(sel & (rev == mi), neg_inf, cur)

    idx_ref[...] = (NSCORE - 1) - revs.T.astype(jnp.int32)
    e = jnp.exp(vals - vals[:1])
    probs_ref[...] = (e / jnp.sum(e, axis=0, keepdims=True)).T


@jax.jit
def kernel(x, W1, W2):
    n_tok = x.shape[0]
    w = jnp.concatenate([W1, W2], axis=0)
    grid = (n_tok // BLK,)
    out = pl.pallas_call(
        _router_kernel,
        grid=grid,
        in_specs=[
            pl.BlockSpec((BLK, D), lambda i: (i, 0)),
            pl.BlockSpec((2 * SK, D), lambda i: (0, 0)),
        ],
        out_specs=[
            pl.BlockSpec((BLK, TOP_K), lambda i: (i, 0)),
            pl.BlockSpec((BLK, TOP_K), lambda i: (i, 0)),
            pl.BlockSpec((BLK, NSCORE), lambda i: (i, 0)),
        ],
        out_shape=[
            jax.ShapeDtypeStruct((n_tok, TOP_K), jnp.int32),
            jax.ShapeDtypeStruct((n_tok, TOP_K), jnp.float32),
            jax.ShapeDtypeStruct((n_tok, NSCORE), jnp.float32),
        ],
    )(x, w)
    return (out[0], out[1], out[2])


# speedup vs baseline: 1.7261x; 1.0276x over previous
import jax
import jax.numpy as jnp
from jax.experimental import pallas as pl
from jax.experimental.pallas import tpu as pltpu

D = 2048
SK = 8
NSCORE = SK * SK
TOP_K = 8
BLK = 1024


def _router_kernel(x_ref, w_ref, idx_ref, probs_ref, scores_ref):
    x = x_ref[...]
    w = w_ref[...]
    s = jax.lax.dot_general(
        x, w, (((1,), (1,)), ((), ())),
        preferred_element_type=jnp.float32)
    st = s.T
    s1t = st[:SK]
    s2t = st[SK:]
    cur = (s1t[:, None, :] + s2t[None, :, :]).reshape(NSCORE, BLK)
    scores_ref[...] = cur.T

    row = jax.lax.broadcasted_iota(jnp.int32, (NSCORE, BLK), 0)
    rev = (NSCORE - 1 - row).astype(jnp.float32)
    row8 = jax.lax.broadcasted_iota(jnp.int32, (TOP_K, BLK), 0)
    vals = jnp.zeros((TOP_K, BLK), jnp.float32)
    revs = jnp.zeros((TOP_K, BLK), jnp.float32)
    neg_one = jnp.float32(-1.0)
    neg_inf = jnp.float32(-jnp.inf)
    for k in range(TOP_K):
        m = jnp.max(cur, axis=0, keepdims=True)
        sel = cur == m
        mi = jnp.max(jnp.where(sel, rev, neg_one), axis=0,
                     keepdims=True)
        vals = jnp.where(row8 == k, m, vals)
        revs = jnp.where(row8 == k, mi, revs)
        cur = jnp.where(sel & (rev == mi), neg_inf, cur)

    idx_ref[...] = (NSCORE - 1) - revs.T.astype(jnp.int32)
    e = jnp.exp(vals - vals[:1])
    probs_ref[...] = (e / jnp.sum(e, axis=0, keepdims=True)).T


@jax.jit
def kernel(x, W1, W2):
    n_tok = x.shape[0]
    w = jnp.concatenate([W1, W2], axis=0)
    grid = (n_tok // BLK,)
    out = pl.pallas_call(
        _router_kernel,
        grid=grid,
        compiler_params=pltpu.CompilerParams(
            dimension_semantics=("parallel",)),
        in_specs=[
            pl.BlockSpec((BLK, D), lambda i: (i, 0)),
            pl.BlockSpec((2 * SK, D), lambda i: (0, 0)),
        ],
        out_specs=[
            pl.BlockSpec((BLK, TOP_K), lambda i: (i, 0)),
            pl.BlockSpec((BLK, TOP_K), lambda i: (i, 0)),
            pl.BlockSpec((BLK, NSCORE), lambda i: (i, 0)),
        ],
        out_shape=[
            jax.ShapeDtypeStruct((n_tok, TOP_K), jnp.int32),
            jax.ShapeDtypeStruct((n_tok, TOP_K), jnp.float32),
            jax.ShapeDtypeStruct((n_tok, NSCORE), jnp.float32),
        ],
    )(x, w)
    return (out[0], out[1], out[2])
